# Initial kernel scaffold; baseline (speedup 1.0000x reference)
import jax
import jax.numpy as jnp
from jax.experimental import pallas as pl


def kernel(x, W_enc, b_enc, W_dec, b_dec):
    raise NotImplementedError("write your pallas kernel here")


# speedup vs baseline: 3.5205x; 3.5205x over previous
import functools

import jax
import jax.numpy as jnp
from jax import lax
from jax.experimental import pallas as pl
from jax.experimental.pallas import tpu as pltpu
from jax.experimental.pallas import tpu_sc as plsc

CH = 128
K = 64


def _enc_body(x_ref, w_ref, benc_ref, bdec_ref, lat_ref, cmax_ref):
    xin = x_ref[...] - bdec_ref[...]
    z = lax.dot_general(
        xin, w_ref[...], (((1,), (1,)), ((), ())),
        preferred_element_type=jnp.float32,
        precision=lax.Precision.DEFAULT,
    )
    z = jnp.maximum(z + benc_ref[...], 0.0)
    lat_ref[...] = z
    bt, bl = z.shape
    for j in range(bl // CH):
        cmax_ref[j:j + 1, :] = jnp.max(z[:, j * CH:(j + 1) * CH], axis=1)[None, :]


def _encoder(x, w_enc, b_enc, b_dec, bt, bl):
    n, d = x.shape
    nl = w_enc.shape[0]
    grid = (nl // bl, n // bt)
    return pl.pallas_call(
        _enc_body,
        grid=grid,
        in_specs=[
            pl.BlockSpec((bt, d), lambda l, t: (t, 0)),
            pl.BlockSpec((bl, d), lambda l, t: (l, 0)),
            pl.BlockSpec((1, bl), lambda l, t: (0, l)),
            pl.BlockSpec((1, d), lambda l, t: (0, 0)),
        ],
        out_specs=[
            pl.BlockSpec((bt, bl), lambda l, t: (t, l)),
            pl.BlockSpec((bl // CH, bt), lambda l, t: (l, t)),
        ],
        out_shape=[
            jax.ShapeDtypeStruct((n, nl), jnp.float32),
            jax.ShapeDtypeStruct((nl // CH, n), jnp.float32),
        ],
    )(x, w_enc, b_enc.reshape(1, nl), b_dec.reshape(1, d))


def _chunk_sel_body(nch, bt, cm_ref, rows_ref, thr_ref):
    t = pl.program_id(0)
    cm = cm_ref[...]
    iota_n = lax.broadcasted_iota(jnp.int32, (nch, bt), 0)
    iota_k = lax.broadcasted_iota(jnp.int32, (K, bt), 0)

    def step(r, carry):
        cm, vals, cids = carry
        m = jnp.max(cm, axis=0, keepdims=True)
        cand = jnp.where(cm == m, iota_n, jnp.int32(2**30))
        cid = jnp.min(cand, axis=0, keepdims=True)
        onr = iota_k == r
        vals = jnp.where(onr, m, vals)
        cids = jnp.where(onr, cid, cids)
        cm = jnp.where(iota_n == cid, -1.0, cm)
        return cm, vals, cids

    init = (cm, jnp.zeros((K, bt), jnp.float32), jnp.zeros((K, bt), jnp.int32))
    _, vals, cids = lax.fori_loop(0, K, step, init)
    tok = t * bt + lax.broadcasted_iota(jnp.int32, (K, bt), 1)
    rows_ref[...] = tok * nch + cids
    thr_ref[...] = vals[K - 1:K, :]


def _chunk_select(cmax, bt):
    nch, n = cmax.shape
    return pl.pallas_call(
        functools.partial(_chunk_sel_body, nch, bt),
        grid=(n // bt,),
        in_specs=[pl.BlockSpec((nch, bt), lambda t: (0, t))],
        out_specs=[
            pl.BlockSpec((K, bt), lambda t: (0, t)),
            pl.BlockSpec((1, bt), lambda t: (0, t)),
        ],
        out_shape=[
            jax.ShapeDtypeStruct((K, n), jnp.int32),
            jax.ShapeDtypeStruct((1, n), jnp.float32),
        ],
    )(cmax)


CMAX = 256


def _sc_gather_filter(lat2d, rows, thr):
    nrows_tab = lat2d.shape[0]
    n = rows.shape[0]
    nch = nrows_tab // n
    nw = 32
    tpw = n // nw
    mesh = plsc.VectorSubcoreMesh(core_axis_name="c", subcore_axis_name="s")
    z16 = jnp.zeros((16,), jnp.int32)
    lane16 = lax.iota(jnp.int32, 16)

    @functools.partial(
        pl.kernel,
        out_type=[
            jax.ShapeDtypeStruct((n, CMAX), jnp.float32),
            jax.ShapeDtypeStruct((n, CMAX), jnp.int32),
        ],
        mesh=mesh,
        scratch_types=[
            pltpu.VMEM((tpw, K), jnp.int32),
            pltpu.VMEM((tpw,), jnp.float32),
            pltpu.VMEM((K, CH), jnp.float32),
            pltpu.VMEM((tpw, CMAX), jnp.float32),
            pltpu.VMEM((tpw, CMAX), jnp.int32),
            pltpu.SemaphoreType.DMA,
        ],
    )
    def c_kernel(lat_hbm, rows_hbm, thr_hbm, outv_hbm, outc_hbm,
                 rows_v, thr_v, buf, ov, oc, sem):
        wid = lax.axis_index("s") * 2 + lax.axis_index("c")
        base = wid * tpw
        pltpu.sync_copy(rows_hbm.at[pl.ds(base, tpw)], rows_v)
        pltpu.sync_copy(thr_hbm.at[pl.ds(base, tpw)], thr_v)

        def per_token(t, _):
            for b in range(CMAX // 16):
                ov[t, pl.ds(b * 16, 16)] = jnp.full((16,), -1.0, jnp.float32)
                oc[t, pl.ds(b * 16, 16)] = z16
            pltpu.async_copy(lat_hbm.at[rows_v.at[t]], buf, sem).wait()
            tvec = plsc.load_gather(thr_v, [z16 + t])

            def chunk_loop(r, off):
                grow = plsc.load_gather(rows_v, [z16 + t, z16 + r])
                colbase = (grow - (base + t) * nch) * CH

                def lane_loop(b, off):
                    v = buf[r, pl.ds(b * 16, 16)]
                    m = v >= tvec
                    gcol = colbase + b * 16 + lane16
                    cnt = lax.reduce_max(
                        plsc.all_reduce_population_count(m), (0,))
                    offc = jnp.minimum(off, CMAX - 16)
                    plsc.store_compressed(ov.at[t, pl.ds(offc, 16)], v, mask=m)
                    plsc.store_compressed(oc.at[t, pl.ds(offc, 16)], gcol,
                                          mask=m)
                    return off + cnt

                return lax.fori_loop(0, CH // 16, lane_loop, off)

            lax.fori_loop(0, K, chunk_loop, jnp.int32(0))
            return 0

        lax.fori_loop(0, tpw, per_token, 0)
        pltpu.sync_copy(ov, outv_hbm.at[pl.ds(base, tpw)])
        pltpu.sync_copy(oc, outc_hbm.at[pl.ds(base, tpw)])

    return c_kernel(lat2d, rows, thr)


def _sc_decode(top_acts, top_idx, w_dec, b_dec):
    n, k = top_acts.shape
    d = w_dec.shape[1]
    nw = 32
    tpw = n // nw
    mesh = plsc.VectorSubcoreMesh(core_axis_name="c", subcore_axis_name="s")
    z16 = jnp.zeros((16,), jnp.int32)

    @functools.partial(
        pl.kernel,
        out_type=jax.ShapeDtypeStruct((n, d), jnp.float32),
        mesh=mesh,
        scratch_types=[
            pltpu.VMEM((tpw, K), jnp.float32),
            pltpu.VMEM((tpw, K), jnp.int32),
            pltpu.VMEM((d,), jnp.float32),
            pltpu.VMEM((K, d), jnp.float32),
            pltpu.VMEM((K, 16), jnp.float32),
            pltpu.VMEM((tpw, d), jnp.float32),
            pltpu.SemaphoreType.DMA,
        ],
    )
    def f_kernel(acts_hbm, idx_hbm, wdec_hbm, bdec_hbm, out_hbm,
                 acts_v, idx_v, bdec_v, buf, abuf, orows, sem):
        wid = lax.axis_index("s") * 2 + lax.axis_index("c")
        base = wid * tpw
        pltpu.sync_copy(acts_hbm.at[pl.ds(base, tpw)], acts_v)
        pltpu.sync_copy(idx_hbm.at[pl.ds(base, tpw)], idx_v)
        pltpu.sync_copy(bdec_hbm, bdec_v)

        def per_token(t, _):
            pltpu.async_copy(wdec_hbm.at[idx_v.at[t]], buf, sem).wait()

            def prep_a(r, _):
                a = plsc.load_gather(acts_v, [z16 + t, z16 + r])
                abuf[r, :] = a.astype(jnp.bfloat16).astype(jnp.float32)
                return 0

            lax.fori_loop(0, K, prep_a, 0)

            def col_loop(b, _):
                sl = pl.ds(b * 16, 16)

                def row_loop(r, acc):
                    w = buf[r, sl].astype(jnp.bfloat16).astype(jnp.float32)
                    return acc + abuf[r, :] * w

                acc = lax.fori_loop(0, K, row_loop, bdec_v[sl])
                orows[t, sl] = acc
                return 0

            lax.fori_loop(0, d // 16, col_loop, 0)
            return 0

        lax.fori_loop(0, tpw, per_token, 0)
        pltpu.sync_copy(orows, out_hbm.at[pl.ds(base, tpw)])

    return f_kernel(top_acts, top_idx, w_dec, b_dec)


def _gather_filter(latents, rows, thr):
    n, nl = latents.shape
    nch = nl // CH
    lat2d = latents.reshape(n * nch, CH)
    chunks = lat2d[rows]
    cols = (rows % nch)[:, :, None] * CH + jnp.arange(CH)[None, None, :]
    vals = chunks.reshape(n, K * CH)
    cols = cols.reshape(n, K * CH)
    mask = vals >= thr
    order = jnp.argsort(~mask, axis=1, stable=True)[:, :2 * CH]
    cand_v = jnp.take_along_axis(jnp.where(mask, vals, -1.0), order, axis=1)
    cand_c = jnp.take_along_axis(cols, order, axis=1)
    return cand_v, cand_c


def _select_body(bt, v_ref, c_ref, acts_ref, idx_ref):
    v = v_ref[...]
    c = c_ref[...]
    iota_k = lax.broadcasted_iota(jnp.int32, (bt, K), 1)

    def step(r, carry):
        v, acts, idx = carry
        m = jnp.max(v, axis=1, keepdims=True)
        key = jnp.where(v == m, c, jnp.int32(2**30))
        cmin = jnp.min(key, axis=1, keepdims=True)
        onr = iota_k == r
        acts = jnp.where(onr, m, acts)
        idx = jnp.where(onr, cmin, idx)
        v = jnp.where(c == cmin, -2.0, v)
        return v, acts, idx

    init = (v, jnp.zeros((bt, K), jnp.float32), jnp.zeros((bt, K), jnp.int32))
    _, acts, idx = lax.fori_loop(0, K, step, init)
    acts_ref[...] = acts
    idx_ref[...] = idx


def _final_select(cand_v, cand_c, bt):
    n, nc = cand_v.shape
    return pl.pallas_call(
        functools.partial(_select_body, bt),
        grid=(n // bt,),
        in_specs=[
            pl.BlockSpec((bt, nc), lambda t: (t, 0)),
            pl.BlockSpec((bt, nc), lambda t: (t, 0)),
        ],
        out_specs=[
            pl.BlockSpec((bt, K), lambda t: (t, 0)),
            pl.BlockSpec((bt, K), lambda t: (t, 0)),
        ],
        out_shape=[
            jax.ShapeDtypeStruct((n, K), jnp.float32),
            jax.ShapeDtypeStruct((n, K), jnp.int32),
        ],
    )(cand_v, cand_c)


def _decode(top_acts, top_idx, w_dec, b_dec):
    rows = w_dec[top_idx]
    return jnp.einsum("nk,nkd->nd", top_acts, rows,
                      precision=lax.Precision.HIGHEST) + b_dec[None, :]


def _loss_body(x_ref, so_ref, out_ref, l2_ref, sx2_ref, sv_ref):
    t = pl.program_id(0)
    nt = pl.num_programs(0)
    x = x_ref[...]
    e = so_ref[...] - x

    @pl.when(t == 0)
    def _():
        l2_ref[0, 0] = 0.0
        sx2_ref[0, 0] = 0.0
        sv_ref[...] = jnp.zeros_like(sv_ref)

    l2_ref[0, 0] += jnp.sum(e * e)
    sx2_ref[0, 0] += jnp.sum(x * x)
    sv_ref[...] += jnp.sum(x, axis=0, keepdims=True)

    @pl.when(t == nt - 1)
    def _():
        sv = sv_ref[...]
        n_total = nt * x.shape[0]
        tv = sx2_ref[0, 0] - jnp.sum(sv * sv) / n_total
        out_ref[...] = (l2_ref[0, 0] / tv) * jnp.ones((1, 1), jnp.float32)


def _loss(x, sae_out, bt):
    n, d = x.shape
    out = pl.pallas_call(
        _loss_body,
        grid=(n // bt,),
        in_specs=[
            pl.BlockSpec((bt, d), lambda t: (t, 0)),
            pl.BlockSpec((bt, d), lambda t: (t, 0)),
        ],
        out_specs=pl.BlockSpec((1, 1), lambda t: (0, 0)),
        out_shape=jax.ShapeDtypeStruct((1, 1), jnp.float32),
        scratch_shapes=[
            pltpu.SMEM((1, 1), jnp.float32),
            pltpu.SMEM((1, 1), jnp.float32),
            pltpu.VMEM((1, d), jnp.float32),
        ],
    )(x, sae_out)
    return out.reshape(())


def kernel(x, W_enc, b_enc, W_dec, b_dec):
    n, d = x.shape
    bt = min(256, n)
    bl = min(2048, W_enc.shape[0])
    latents, cmax = _encoder(x, W_enc, b_enc, b_dec, bt, bl)
    rows_t, thr_t = _chunk_select(cmax, min(512, n))
    rows, thr = rows_t.T, thr_t.T
    cand_v, cand_c = _gather_filter(latents, rows, thr)
    top_acts, top_idx = _final_select(cand_v, cand_c, min(512, n))
    sae_out = _decode(top_acts, top_idx, W_dec, b_dec)
    loss = _loss(x, sae_out, min(256, n))
    return (sae_out, top_acts, top_idx, loss)


# speedup vs baseline: 5.7341x; 1.6288x over previous
import functools

import jax
import jax.numpy as jnp
from jax import lax
from jax.experimental import pallas as pl
from jax.experimental.pallas import tpu as pltpu
from jax.experimental.pallas import tpu_sc as plsc

CH = 32
K = 64


def _enc_body(x_ref, w_ref, benc_ref, bdec_ref, lat_ref, cmax_ref):
    xin = x_ref[...] - bdec_ref[...]
    z = lax.dot_general(
        xin, w_ref[...], (((1,), (1,)), ((), ())),
        preferred_element_type=jnp.float32,
        precision=lax.Precision.DEFAULT,
    )
    z = jnp.maximum(z + benc_ref[...], 0.0)
    lat_ref[...] = z
    bt, bl = z.shape
    cmax_ref[...] = jnp.max(z.reshape(bt, bl // CH, CH), axis=2)


def _encoder(x, w_enc, b_enc, b_dec, bt, bl):
    n, d = x.shape
    nl = w_enc.shape[0]
    grid = (nl // bl, n // bt)
    return pl.pallas_call(
        _enc_body,
        grid=grid,
        in_specs=[
            pl.BlockSpec((bt, d), lambda l, t: (t, 0)),
            pl.BlockSpec((bl, d), lambda l, t: (l, 0)),
            pl.BlockSpec((1, bl), lambda l, t: (0, l)),
            pl.BlockSpec((1, d), lambda l, t: (0, 0)),
        ],
        out_specs=[
            pl.BlockSpec((bt, bl), lambda l, t: (t, l)),
            pl.BlockSpec((bt, bl // CH), lambda l, t: (t, l)),
        ],
        out_shape=[
            jax.ShapeDtypeStruct((n, nl), jnp.float32),
            jax.ShapeDtypeStruct((n, nl // CH), jnp.float32),
        ],
    )(x, w_enc, b_enc.reshape(1, nl), b_dec.reshape(1, d))


def _chunk_sel_body(nch, bt, cm_ref, cids_ref):
    cm = cm_ref[...]
    iota_n = lax.broadcasted_iota(jnp.int32, (bt, nch), 1)
    iota_k = lax.broadcasted_iota(jnp.int32, (bt, K), 1)

    def step(r, carry):
        cm, cids = carry
        m = jnp.max(cm, axis=1, keepdims=True)
        cand = jnp.where(cm == m, iota_n, jnp.int32(2**30))
        cid = jnp.min(cand, axis=1, keepdims=True)
        cids = jnp.where(iota_k == r, cid, cids)
        cm = jnp.where(iota_n == cid, -1.0, cm)
        return cm, cids

    init = (cm, jnp.zeros((bt, K), jnp.int32))
    _, cids = lax.fori_loop(0, K, step, init)
    cids_ref[...] = cids


def _chunk_select(cmax, bt):
    n, nch = cmax.shape
    return pl.pallas_call(
        functools.partial(_chunk_sel_body, nch, bt),
        grid=(n // bt,),
        in_specs=[pl.BlockSpec((bt, nch), lambda t: (t, 0))],
        out_specs=pl.BlockSpec((bt, K), lambda t: (t, 0)),
        out_shape=jax.ShapeDtypeStruct((n, K), jnp.int32),
    )(cmax)


TG = 2


def _sc_gather(lat2d, subs):
    n = subs.shape[0]
    npar = lat2d.shape[0] // n
    nw = 32
    tpw = n // nw
    ng = tpw // TG
    mesh = plsc.VectorSubcoreMesh(core_axis_name="c", subcore_axis_name="s")

    @functools.partial(
        pl.kernel,
        out_type=jax.ShapeDtypeStruct((n * K, 128), jnp.float32),
        mesh=mesh,
        scratch_types=[
            pltpu.VMEM((tpw, K), jnp.int32),
            pltpu.VMEM((TG * K,), jnp.int32),
            pltpu.VMEM((TG * K, 128), jnp.float32),
            pltpu.SemaphoreType.DMA,
        ],
    )
    def c_kernel(lat_hbm, subs_hbm, out_hbm, subs_v, idx_v, buf, sem):
        wid = lax.axis_index("s") * 2 + lax.axis_index("c")
        base = wid * tpw
        pltpu.sync_copy(subs_hbm.at[pl.ds(base, tpw)], subs_v)

        def per_group(g, _):
            def build_idx(t, _):
                tok = base + g * TG + t

                def build_row(rv, _):
                    c16 = subs_v[g * TG + t, pl.ds(rv * 16, 16)]
                    idx_v[pl.ds(t * K + rv * 16, 16)] = (
                        (c16 >> 2) + tok * npar)
                    return 0

                lax.fori_loop(0, K // 16, build_row, 0)
                return 0

            lax.fori_loop(0, TG, build_idx, 0)
            pltpu.async_copy(lat_hbm.at[idx_v], buf, sem).wait()
            pltpu.sync_copy(buf,
                            out_hbm.at[pl.ds((base + g * TG) * K, TG * K)])
            return 0

        lax.fori_loop(0, ng, per_group, 0)

    return c_kernel(lat2d, subs)


def _bcast(vec, lane):
    idx = jnp.zeros((16,), jnp.int32) + lane
    return lax.gather(
        vec, idx[:, None],
        lax.GatherDimensionNumbers(offset_dims=(), collapsed_slice_dims=(0,),
                                   start_index_map=(0,)),
        (1,), mode=lax.GatherScatterMode.PROMISE_IN_BOUNDS)


def _sc_decode(top_acts, top_idx, w_dec, b_dec):
    n, k = top_acts.shape
    d = w_dec.shape[1]
    nw = 32
    tpw = n // nw
    mesh = plsc.VectorSubcoreMesh(core_axis_name="c", subcore_axis_name="s")

    @functools.partial(
        pl.kernel,
        out_type=jax.ShapeDtypeStruct((n * d,), jnp.float32),
        mesh=mesh,
        scratch_types=[
            pltpu.VMEM((tpw, K), jnp.float32),
            pltpu.VMEM((tpw, K), jnp.int32),
            pltpu.VMEM((d,), jnp.float32),
            pltpu.VMEM((K, d), jnp.float32),
            pltpu.VMEM((K * 16,), jnp.float32),
            pltpu.VMEM((16 * d,), jnp.float32),
            pltpu.SemaphoreType.DMA,
        ],
    )
    def f_kernel(acts_hbm, idx_hbm, wdec_hbm, bdec_hbm, out_hbm,
                 acts_v, idx_v, bdec_v, buf, abuf, orows, sem):
        z16f = jnp.zeros((16,), jnp.float32)
        lane16 = lax.iota(jnp.int32, 16)
        wid = lax.axis_index("s") * 2 + lax.axis_index("c")
        base = wid * tpw
        pltpu.sync_copy(acts_hbm.at[pl.ds(base, tpw)], acts_v)
        pltpu.sync_copy(idx_hbm.at[pl.ds(base, tpw)], idx_v)
        pltpu.sync_copy(bdec_hbm, bdec_v)

        def per_group(g, _):
            def per_token(tt, _):
                t = g * 16 + tt
                pltpu.async_copy(wdec_hbm.at[idx_v.at[t]], buf, sem).wait()

                def prep_a(r, _):
                    a16 = acts_v[t, pl.ds((r // 16) * 16, 16)]
                    abuf[pl.ds(r * 16, 16)] = _bcast(a16, r % 16)
                    return 0

                lax.fori_loop(0, K, prep_a, 0)

                def col_loop(b, _):
                    def row_loop(r, acc):
                        w = buf[r, pl.ds(b * 16, 16)]
                        return acc + abuf[pl.ds(r * 16, 16)] * w

                    acc = lax.fori_loop(0, K, row_loop,
                                        bdec_v[pl.ds(b * 16, 16)])
                    orows[pl.ds(tt * d + b * 16, 16)] = acc
                    return 0

                lax.fori_loop(0, d // 16, col_loop, 0)
                return 0

            lax.fori_loop(0, 16, per_token, 0)
            pltpu.sync_copy(orows,
                            out_hbm.at[pl.ds((base + g * 16) * d, 16 * d)])
            return 0

        lax.fori_loop(0, tpw // 16, per_group, 0)

    return f_kernel(top_acts, top_idx, w_dec, b_dec).reshape(n, d)


def _select_body(bt, pr_ref, subs_ref, acts_ref, idx_ref):
    parents = pr_ref[...].reshape(bt, K, 128)
    subs = subs_ref[...]
    nc = K * CH
    subm4 = (subs % 4)[:, :, None]
    ext = jnp.zeros((bt, K, CH), jnp.float32)
    for j in range(4):
        ext = ext + jnp.where(subm4 == j,
                              parents[:, :, j * CH:(j + 1) * CH], 0.0)
    v = ext.reshape(bt, nc)
    iota_kk = lax.broadcasted_iota(jnp.int32, (K, nc), 0)
    iota_nc = lax.broadcasted_iota(jnp.int32, (K, nc), 1)
    expand = jnp.where(iota_kk == iota_nc // CH, 1.0, 0.0)
    sexp = lax.dot_general(
        subs.astype(jnp.float32), expand,
        (((1,), (0,)), ((), ())),
        preferred_element_type=jnp.float32,
        precision=lax.Precision.HIGHEST,
    )
    lane_in_chunk = lax.broadcasted_iota(jnp.int32, (bt, nc), 1) % CH
    c = sexp.astype(jnp.int32) * CH + lane_in_chunk
    iota_k = lax.broadcasted_iota(jnp.int32, (bt, K), 1)

    def step(r, carry):
        v, acts, idx = carry
        m = jnp.max(v, axis=1, keepdims=True)
        key = jnp.where(v == m, c, jnp.int32(2**30))
        cmin = jnp.min(key, axis=1, keepdims=True)
        onr = iota_k == r
        acts = jnp.where(onr, m, acts)
        idx = jnp.where(onr, cmin, idx)
        v = jnp.where(c == cmin, -2.0, v)
        return v, acts, idx

    init = (v, jnp.zeros((bt, K), jnp.float32), jnp.zeros((bt, K), jnp.int32))
    _, acts, idx = lax.fori_loop(0, K, step, init)
    acts_ref[...] = acts
    idx_ref[...] = idx


def _final_select(parent_rows, subs, bt):
    n = subs.shape[0]
    return pl.pallas_call(
        functools.partial(_select_body, bt),
        grid=(n // bt,),
        in_specs=[
            pl.BlockSpec((bt, K * 128), lambda t: (t, 0)),
            pl.BlockSpec((bt, K), lambda t: (t, 0)),
        ],
        out_specs=[
            pl.BlockSpec((bt, K), lambda t: (t, 0)),
            pl.BlockSpec((bt, K), lambda t: (t, 0)),
        ],
        out_shape=[
            jax.ShapeDtypeStruct((n, K), jnp.float32),
            jax.ShapeDtypeStruct((n, K), jnp.int32),
        ],
    )(parent_rows, subs)


def _loss_body(x_ref, so_ref, out_ref, l2_ref, sx2_ref, sv_ref):
    t = pl.program_id(0)
    nt = pl.num_programs(0)
    x = x_ref[...]
    e = so_ref[...] - x

    @pl.when(t == 0)
    def _():
        l2_ref[0, 0] = 0.0
        sx2_ref[0, 0] = 0.0
        sv_ref[...] = jnp.zeros_like(sv_ref)

    l2_ref[0, 0] += jnp.sum(e * e)
    sx2_ref[0, 0] += jnp.sum(x * x)
    sv_ref[...] += jnp.sum(x, axis=0, keepdims=True)

    @pl.when(t == nt - 1)
    def _():
        sv = sv_ref[...]
        n_total = nt * x.shape[0]
        tv = sx2_ref[0, 0] - jnp.sum(sv * sv) / n_total
        out_ref[...] = (l2_ref[0, 0] / tv) * jnp.ones((1, 1), jnp.float32)


def _loss(x, sae_out, bt):
    n, d = x.shape
    out = pl.pallas_call(
        _loss_body,
        grid=(n // bt,),
        in_specs=[
            pl.BlockSpec((bt, d), lambda t: (t, 0)),
            pl.BlockSpec((bt, d), lambda t: (t, 0)),
        ],
        out_specs=pl.BlockSpec((1, 1), lambda t: (0, 0)),
        out_shape=jax.ShapeDtypeStruct((1, 1), jnp.float32),
        scratch_shapes=[
            pltpu.SMEM((1, 1), jnp.float32),
            pltpu.SMEM((1, 1), jnp.float32),
            pltpu.VMEM((1, d), jnp.float32),
        ],
    )(x, sae_out)
    return out.reshape(())


def kernel(x, W_enc, b_enc, W_dec, b_dec):
    n, d = x.shape
    nl = W_enc.shape[0]
    bt = min(256, n)
    bl = min(4096, nl)
    latents, smax = _encoder(x, W_enc, b_enc, b_dec, bt, bl)
    subs = _chunk_select(smax, min(512, n))
    lat2d = latents.reshape(n * (nl // 128), 128)
    parent_rows = _sc_gather(lat2d, subs).reshape(n, K * 128)
    top_acts, top_idx = _final_select(parent_rows, subs, min(256, n))
    sae_out = _sc_decode(top_acts, top_idx, W_dec, b_dec)
    loss = _loss(x, sae_out, min(256, n))
    return (sae_out, top_acts, top_idx, loss)
